# baseline (speedup 1.0000x reference)
import functools

import jax
import jax.numpy as jnp
from jax import lax
from jax.experimental import pallas as pl
from jax.experimental.pallas import tpu as pltpu
from jax.experimental.pallas import tpu_sc as plsc

N = 10000
E = 320000
D_IN = 128
D_HID = 128
N_CLASS = 64

NC = 2
NS = 16
NW = NC * NS
CHUNK = 128
NCW = 160
E_PAD = NS * NCW * CHUNK
PAD_NODE = N

N_PAD = 10112
ROWS_PER_TILE = N_PAD // NS
NZCH = ROWS_PER_TILE // CHUNK
TAIL = ROWS_PER_TILE - NZCH * CHUNK
BR = 128

_MESH = plsc.VectorSubcoreMesh(core_axis_name="c", subcore_axis_name="s",
                               num_cores=1)


def _make_agg(d):
    @functools.partial(
        pl.kernel,
        out_type=jax.ShapeDtypeStruct((N_PAD, d), jnp.float32),
        mesh=_MESH,
        scratch_types=[
            pltpu.VMEM((8, CHUNK), jnp.int32),
            pltpu.VMEM((2, CHUNK, d), jnp.float32),
            pltpu.VMEM_SHARED((N_PAD, d), jnp.float32),
            pltpu.SemaphoreType.DMA,
            pltpu.SemaphoreType.DMA,
            pltpu.SemaphoreType.DMA,
            pltpu.SemaphoreType.DMA,
            pltpu.SemaphoreType.DMA,
        ],
    )
    def agg(xw_hbm, src_hbm, dst_hbm, zeros_hbm, out_hbm,
            ix_v, rows_v, acc_sh, gsem_a, gsem_b, isem_a, isem_b, csem):
        wid = lax.axis_index("s")
        base = wid * ROWS_PER_TILE

        pltpu.async_copy(zeros_hbm, rows_v.at[0], csem).wait()
        for r in range(NZCH):
            pltpu.sync_copy(rows_v.at[0], acc_sh.at[pl.ds(base + r * CHUNK, CHUNK)])
        pltpu.sync_copy(rows_v.at[0, pl.ds(0, TAIL)],
                        acc_sh.at[pl.ds(base + NZCH * CHUNK, TAIL)])

        plsc.subcore_barrier()

        @pl.loop(0, NCW, step=4)
        def _(j):
            hia = [pltpu.async_copy(src_hbm.at[wid, pl.ds(j, 2)],
                                    ix_v.at[pl.ds(0, 2)], isem_a),
                   pltpu.async_copy(dst_hbm.at[wid, pl.ds(j, 2)],
                                    ix_v.at[pl.ds(2, 2)], isem_a)]
            hib = [pltpu.async_copy(src_hbm.at[wid, pl.ds(j + 2, 2)],
                                    ix_v.at[pl.ds(4, 2)], isem_b),
                   pltpu.async_copy(dst_hbm.at[wid, pl.ds(j + 2, 2)],
                                    ix_v.at[pl.ds(6, 2)], isem_b)]
            for h in hia:
                h.wait()
            ga = pltpu.async_copy(xw_hbm.at[ix_v.at[0]], rows_v.at[0], gsem_a)
            ga.wait()
            gb = pltpu.async_copy(xw_hbm.at[ix_v.at[1]], rows_v.at[1], gsem_b)
            pltpu.sync_copy(rows_v.at[0], acc_sh.at[ix_v.at[2]], add=True)
            gb.wait()
            for h in hib:
                h.wait()
            ga = pltpu.async_copy(xw_hbm.at[ix_v.at[4]], rows_v.at[0], gsem_a)
            pltpu.sync_copy(rows_v.at[1], acc_sh.at[ix_v.at[3]], add=True)
            ga.wait()
            gb = pltpu.async_copy(xw_hbm.at[ix_v.at[5]], rows_v.at[1], gsem_b)
            pltpu.sync_copy(rows_v.at[0], acc_sh.at[ix_v.at[6]], add=True)
            gb.wait()
            pltpu.sync_copy(rows_v.at[1], acc_sh.at[ix_v.at[7]], add=True)

        plsc.subcore_barrier()

        ssems = (gsem_a, gsem_b)
        store_h = [None, None]
        for i, ln in enumerate([CHUNK] * NZCH + [TAIL]):
            b = i % 2
            off = base + i * CHUNK
            if store_h[b] is not None:
                store_h[b].wait()
            pltpu.async_copy(acc_sh.at[pl.ds(off, ln)],
                             rows_v.at[b, pl.ds(0, ln)], csem).wait()
            store_h[b] = pltpu.async_copy(rows_v.at[b, pl.ds(0, ln)],
                                          out_hbm.at[pl.ds(off, ln)],
                                          ssems[b])
        for h in store_h:
            h.wait()

    return agg


_agg_hid = _make_agg(D_HID)


@functools.partial(
    pl.kernel,
    out_type=jax.ShapeDtypeStruct((N_PAD,), jnp.float32),
    mesh=_MESH,
    scratch_types=[
        pltpu.VMEM((NCW, CHUNK), jnp.int32),
        pltpu.VMEM((2, CHUNK), jnp.float32),
        pltpu.VMEM((NZCH + 1, CHUNK), jnp.float32),
        pltpu.VMEM_SHARED((N_PAD,), jnp.float32),
        pltpu.SemaphoreType.DMA,
        pltpu.SemaphoreType.DMA,
    ],
)
def _deg_kernel(dst_hbm, zo_hbm, out_hbm, idx_v, zo_v, stage_v, acc_sh,
                ssem, csem):
    wid = lax.axis_index("s")

    h1 = pltpu.async_copy(dst_hbm.at[wid], idx_v, csem)
    h2 = pltpu.async_copy(zo_hbm, zo_v, csem)
    h1.wait()
    h2.wait()

    for r in range(NZCH):
        pltpu.sync_copy(zo_v.at[0],
                        acc_sh.at[pl.ds(wid * ROWS_PER_TILE + r * CHUNK, CHUNK)])
    pltpu.sync_copy(zo_v.at[0, pl.ds(0, TAIL)],
                    acc_sh.at[pl.ds(wid * ROWS_PER_TILE + NZCH * CHUNK, TAIL)])

    plsc.subcore_barrier()

    @pl.loop(0, NCW)
    def _(j):
        pltpu.async_copy(zo_v.at[1], acc_sh.at[idx_v.at[j]], ssem, add=True)

    @pl.loop(0, NCW)
    def _(j):
        pltpu.make_async_copy(zo_v.at[1], acc_sh.at[idx_v.at[0]], ssem).wait()

    plsc.subcore_barrier()

    tbase = wid * ROWS_PER_TILE + NZCH * CHUNK
    loads = [pltpu.async_copy(
                 acc_sh.at[pl.ds(wid * ROWS_PER_TILE + r * CHUNK, CHUNK)],
                 stage_v.at[r], csem) for r in range(NZCH)]
    loads.append(pltpu.async_copy(acc_sh.at[pl.ds(tbase, TAIL)],
                                  stage_v.at[NZCH, pl.ds(0, TAIL)], csem))
    for h in loads:
        h.wait()
    stores = [pltpu.async_copy(
                  stage_v.at[r],
                  out_hbm.at[pl.ds(wid * ROWS_PER_TILE + r * CHUNK, CHUNK)],
                  csem) for r in range(NZCH)]
    stores.append(pltpu.async_copy(stage_v.at[NZCH, pl.ds(0, TAIL)],
                                   out_hbm.at[pl.ds(tbase, TAIL)], csem))
    for h in stores:
        h.wait()


def _scale1_body(x_ref, w_ref, d_ref, xws_ref, dis_ref):
    deg = d_ref[...] + 1.0
    dis = lax.rsqrt(deg)
    xw = jnp.dot(x_ref[...], w_ref[...], precision=lax.Precision.HIGHEST,
                 preferred_element_type=jnp.float32)
    xws_ref[...] = xw * dis
    dis_ref[...] = dis


def _layer1(x_pad, W1, d0):
    return pl.pallas_call(
        _scale1_body,
        grid=(N_PAD // BR,),
        in_specs=[
            pl.BlockSpec((BR, D_IN), lambda i: (i, 0)),
            pl.BlockSpec((D_IN, D_HID), lambda i: (0, 0)),
            pl.BlockSpec((BR, 1), lambda i: (i, 0)),
        ],
        out_specs=[
            pl.BlockSpec((BR, D_HID), lambda i: (i, 0)),
            pl.BlockSpec((BR, 1), lambda i: (i, 0)),
        ],
        out_shape=[
            jax.ShapeDtypeStruct((N_PAD, D_HID), jnp.float32),
            jax.ShapeDtypeStruct((N_PAD, 1), jnp.float32),
        ],
    )(x_pad, W1, d0)


def _layer2_body(p0_ref, xws_ref, dis_ref, b1_ref, w2_ref, out_ref):
    s = p0_ref[...] + xws_ref[...]
    pre = dis_ref[...] * s + b1_ref[...]
    h = jnp.maximum(pre, 0.0)
    xw2 = jnp.dot(h, w2_ref[...], precision=lax.Precision.HIGHEST,
                  preferred_element_type=jnp.float32)
    xw2s = xw2 * dis_ref[...]
    out_ref[...] = jnp.concatenate([xw2s, jnp.zeros_like(xw2s)], axis=1)


def _layer2(p0, xw1s, dis2d, b1, W2):
    return pl.pallas_call(
        _layer2_body,
        grid=(N_PAD // BR,),
        in_specs=[
            pl.BlockSpec((BR, D_HID), lambda i: (i, 0)),
            pl.BlockSpec((BR, D_HID), lambda i: (i, 0)),
            pl.BlockSpec((BR, 1), lambda i: (i, 0)),
            pl.BlockSpec((1, D_HID), lambda i: (0, 0)),
            pl.BlockSpec((D_HID, N_CLASS), lambda i: (0, 0)),
        ],
        out_specs=pl.BlockSpec((BR, 2 * N_CLASS), lambda i: (i, 0)),
        out_shape=jax.ShapeDtypeStruct((N_PAD, 2 * N_CLASS), jnp.float32),
    )(p0, xw1s, dis2d, b1, W2)


def _final_body(p0_ref, xws_ref, dis_ref, b2_ref, lsm_ref, sm_ref):
    s = (p0_ref[...] + xws_ref[...])[:, :N_CLASS]
    logits = dis_ref[...] * s + b2_ref[...]
    m = jnp.max(logits, axis=1, keepdims=True)
    sh = logits - m
    ex = jnp.exp(sh)
    z = jnp.sum(ex, axis=1, keepdims=True)
    lsm_ref[...] = sh - jnp.log(z)
    sm_ref[...] = ex / z


def _final(p0, xw2s, dis2d, b2):
    return pl.pallas_call(
        _final_body,
        grid=(N_PAD // BR,),
        in_specs=[
            pl.BlockSpec((BR, 2 * N_CLASS), lambda i: (i, 0)),
            pl.BlockSpec((BR, 2 * N_CLASS), lambda i: (i, 0)),
            pl.BlockSpec((BR, 1), lambda i: (i, 0)),
            pl.BlockSpec((1, N_CLASS), lambda i: (0, 0)),
        ],
        out_specs=[
            pl.BlockSpec((BR, N_CLASS), lambda i: (i, 0)),
            pl.BlockSpec((BR, N_CLASS), lambda i: (i, 0)),
        ],
        out_shape=[
            jax.ShapeDtypeStruct((N_PAD, N_CLASS), jnp.float32),
            jax.ShapeDtypeStruct((N_PAD, N_CLASS), jnp.float32),
        ],
    )(p0, xw2s, dis2d, b2)


def kernel(features, edge_index, W1, b1, W2, b2):
    src = edge_index[0].astype(jnp.int32)
    dst = edge_index[1].astype(jnp.int32)
    pad = jnp.full((E_PAD - E,), PAD_NODE, jnp.int32)
    src_p = jnp.concatenate([src, pad]).reshape(NS, NCW, CHUNK)
    dst_p = jnp.concatenate([dst, pad]).reshape(NS, NCW, CHUNK)
    x_pad = jnp.pad(features, ((0, N_PAD - N), (0, 0)))

    zo = jnp.stack([jnp.zeros((CHUNK,), jnp.float32),
                    jnp.ones((CHUNK,), jnp.float32)])
    deg = _deg_kernel(dst_p, zo)
    d0 = deg.reshape(N_PAD, 1)

    xw1s, dis2d = _layer1(x_pad, W1, d0)

    z_hid = jnp.zeros((CHUNK, D_HID), jnp.float32)
    s1 = _agg_hid(xw1s, src_p, dst_p, z_hid)

    xw2s = _layer2(s1, xw1s, dis2d, b1.reshape(1, D_HID), W2)

    s2 = _agg_hid(xw2s, src_p, dst_p, z_hid)

    lsm, sm = _final(s2, xw2s, dis2d, b2.reshape(1, N_CLASS))
    return lsm[:N], sm[:N]


# speedup vs baseline: 1.2709x; 1.2709x over previous
import functools

import jax
import jax.numpy as jnp
from jax import lax
from jax.experimental import pallas as pl
from jax.experimental.pallas import tpu as pltpu
from jax.experimental.pallas import tpu_sc as plsc

N = 10000
E = 320000
D_IN = 128
D_HID = 128
N_CLASS = 64

NC = 2
NS = 16
NW = NC * NS
CHUNK = 128
C0 = 128
C1 = 32
NCW = max(C0, C1)
N_USED_CHUNKS = NS * (C0 + C1)
E_PAD = N_USED_CHUNKS * CHUNK
PAD_NODE = N

N_PAD = 10112
ROWS_PER_TILE = N_PAD // NS
NZCH = ROWS_PER_TILE // CHUNK
TAIL = ROWS_PER_TILE - NZCH * CHUNK
BR = 128

_MESH = plsc.VectorSubcoreMesh(core_axis_name="c", subcore_axis_name="s")


def _make_agg(d):
    @functools.partial(
        pl.kernel,
        out_type=jax.ShapeDtypeStruct((NC, N_PAD, d), jnp.float32),
        mesh=_MESH,
        scratch_types=[
            pltpu.VMEM((8, CHUNK), jnp.int32),
            pltpu.VMEM((2, CHUNK, d), jnp.float32),
            pltpu.VMEM_SHARED((N_PAD, d), jnp.float32),
            pltpu.SemaphoreType.DMA,
            pltpu.SemaphoreType.DMA,
            pltpu.SemaphoreType.DMA,
            pltpu.SemaphoreType.DMA,
            pltpu.SemaphoreType.DMA,
        ],
    )
    def agg(xw_hbm, src_hbm, dst_hbm, zeros_hbm, out_hbm,
            ix_v, rows_v, acc_sh, gsem_a, gsem_b, isem_a, isem_b, csem):
        cid = lax.axis_index("c")
        sid = lax.axis_index("s")
        wid = sid * NC + cid
        base = sid * ROWS_PER_TILE

        pltpu.async_copy(zeros_hbm, rows_v.at[0], csem).wait()
        for r in range(NZCH):
            pltpu.sync_copy(rows_v.at[0], acc_sh.at[pl.ds(base + r * CHUNK, CHUNK)])
        pltpu.sync_copy(rows_v.at[0, pl.ds(0, TAIL)],
                        acc_sh.at[pl.ds(base + NZCH * CHUNK, TAIL)])

        plsc.subcore_barrier()

        cnt = jnp.where(cid == 0, C0, C1)

        @pl.loop(0, cnt, step=4)
        def _(j):
            hia = [pltpu.async_copy(src_hbm.at[wid, pl.ds(j, 2)],
                                    ix_v.at[pl.ds(0, 2)], isem_a),
                   pltpu.async_copy(dst_hbm.at[wid, pl.ds(j, 2)],
                                    ix_v.at[pl.ds(2, 2)], isem_a)]
            hib = [pltpu.async_copy(src_hbm.at[wid, pl.ds(j + 2, 2)],
                                    ix_v.at[pl.ds(4, 2)], isem_b),
                   pltpu.async_copy(dst_hbm.at[wid, pl.ds(j + 2, 2)],
                                    ix_v.at[pl.ds(6, 2)], isem_b)]
            for h in hia:
                h.wait()
            ga = pltpu.async_copy(xw_hbm.at[ix_v.at[0]], rows_v.at[0], gsem_a)
            ga.wait()
            gb = pltpu.async_copy(xw_hbm.at[ix_v.at[1]], rows_v.at[1], gsem_b)
            pltpu.sync_copy(rows_v.at[0], acc_sh.at[ix_v.at[2]], add=True)
            gb.wait()
            for h in hib:
                h.wait()
            ga = pltpu.async_copy(xw_hbm.at[ix_v.at[4]], rows_v.at[0], gsem_a)
            pltpu.sync_copy(rows_v.at[1], acc_sh.at[ix_v.at[3]], add=True)
            ga.wait()
            gb = pltpu.async_copy(xw_hbm.at[ix_v.at[5]], rows_v.at[1], gsem_b)
            pltpu.sync_copy(rows_v.at[0], acc_sh.at[ix_v.at[6]], add=True)
            gb.wait()
            pltpu.sync_copy(rows_v.at[1], acc_sh.at[ix_v.at[7]], add=True)

        plsc.subcore_barrier()

        ssems = (gsem_a, gsem_b)
        store_h = [None, None]
        for i, ln in enumerate([CHUNK] * NZCH + [TAIL]):
            b = i % 2
            off = base + i * CHUNK
            if store_h[b] is not None:
                store_h[b].wait()
            pltpu.async_copy(acc_sh.at[pl.ds(off, ln)],
                             rows_v.at[b, pl.ds(0, ln)], csem).wait()
            store_h[b] = pltpu.async_copy(rows_v.at[b, pl.ds(0, ln)],
                                          out_hbm.at[cid, pl.ds(off, ln)],
                                          ssems[b])
        for h in store_h:
            h.wait()

    return agg


_agg_hid = _make_agg(D_HID)


@functools.partial(
    pl.kernel,
    out_type=jax.ShapeDtypeStruct((NC * N_PAD,), jnp.float32),
    mesh=_MESH,
    scratch_types=[
        pltpu.VMEM((NCW, CHUNK), jnp.int32),
        pltpu.VMEM((2, CHUNK), jnp.float32),
        pltpu.VMEM((NZCH + 1, CHUNK), jnp.float32),
        pltpu.VMEM_SHARED((N_PAD,), jnp.float32),
        pltpu.SemaphoreType.DMA,
        pltpu.SemaphoreType.DMA,
    ],
)
def _deg_kernel(dst_hbm, zo_hbm, out_hbm, idx_v, zo_v, stage_v, acc_sh,
                ssem, csem):
    cid = lax.axis_index("c")
    sid = lax.axis_index("s")
    wid = sid * NC + cid

    h1 = pltpu.async_copy(dst_hbm.at[wid], idx_v, csem)
    h2 = pltpu.async_copy(zo_hbm, zo_v, csem)
    h1.wait()
    h2.wait()

    for r in range(NZCH):
        pltpu.sync_copy(zo_v.at[0],
                        acc_sh.at[pl.ds(sid * ROWS_PER_TILE + r * CHUNK, CHUNK)])
    pltpu.sync_copy(zo_v.at[0, pl.ds(0, TAIL)],
                    acc_sh.at[pl.ds(sid * ROWS_PER_TILE + NZCH * CHUNK, TAIL)])

    plsc.subcore_barrier()

    cnt = jnp.where(cid == 0, C0, C1)

    @pl.loop(0, cnt)
    def _(j):
        pltpu.async_copy(zo_v.at[1], acc_sh.at[idx_v.at[j]], ssem, add=True)

    @pl.loop(0, cnt)
    def _(j):
        pltpu.make_async_copy(zo_v.at[1], acc_sh.at[idx_v.at[0]], ssem).wait()

    plsc.subcore_barrier()

    tbase = sid * ROWS_PER_TILE + NZCH * CHUNK
    loads = [pltpu.async_copy(
                 acc_sh.at[pl.ds(sid * ROWS_PER_TILE + r * CHUNK, CHUNK)],
                 stage_v.at[r], csem) for r in range(NZCH)]
    loads.append(pltpu.async_copy(acc_sh.at[pl.ds(tbase, TAIL)],
                                  stage_v.at[NZCH, pl.ds(0, TAIL)], csem))
    for h in loads:
        h.wait()
    stores = [pltpu.async_copy(
                  stage_v.at[r],
                  out_hbm.at[pl.ds(cid * N_PAD + sid * ROWS_PER_TILE + r * CHUNK,
                                   CHUNK)],
                  csem) for r in range(NZCH)]
    stores.append(pltpu.async_copy(stage_v.at[NZCH, pl.ds(0, TAIL)],
                                   out_hbm.at[pl.ds(cid * N_PAD + tbase, TAIL)],
                                   csem))
    for h in stores:
        h.wait()


def _scale1_body(x_ref, w_ref, d0_ref, d1_ref, xws_ref, dis_ref):
    deg = d0_ref[...] + d1_ref[...] + 1.0
    dis = lax.rsqrt(deg)
    xw = jnp.dot(x_ref[...], w_ref[...], precision=lax.Precision.HIGHEST,
                 preferred_element_type=jnp.float32)
    xws_ref[...] = xw * dis
    dis_ref[...] = dis


def _layer1(x_pad, W1, d0, d1):
    return pl.pallas_call(
        _scale1_body,
        grid=(N_PAD // BR,),
        in_specs=[
            pl.BlockSpec((BR, D_IN), lambda i: (i, 0)),
            pl.BlockSpec((D_IN, D_HID), lambda i: (0, 0)),
            pl.BlockSpec((BR, 1), lambda i: (i, 0)),
            pl.BlockSpec((BR, 1), lambda i: (i, 0)),
        ],
        out_specs=[
            pl.BlockSpec((BR, D_HID), lambda i: (i, 0)),
            pl.BlockSpec((BR, 1), lambda i: (i, 0)),
        ],
        out_shape=[
            jax.ShapeDtypeStruct((N_PAD, D_HID), jnp.float32),
            jax.ShapeDtypeStruct((N_PAD, 1), jnp.float32),
        ],
    )(x_pad, W1, d0, d1)


def _layer2_body(p0_ref, p1_ref, xws_ref, dis_ref, b1_ref, w2_ref, out_ref):
    s = p0_ref[...] + p1_ref[...] + xws_ref[...]
    pre = dis_ref[...] * s + b1_ref[...]
    h = jnp.maximum(pre, 0.0)
    xw2 = jnp.dot(h, w2_ref[...], precision=lax.Precision.HIGHEST,
                  preferred_element_type=jnp.float32)
    xw2s = xw2 * dis_ref[...]
    out_ref[...] = jnp.concatenate([xw2s, jnp.zeros_like(xw2s)], axis=1)


def _layer2(p0, p1, xw1s, dis2d, b1, W2):
    return pl.pallas_call(
        _layer2_body,
        grid=(N_PAD // BR,),
        in_specs=[
            pl.BlockSpec((BR, D_HID), lambda i: (i, 0)),
            pl.BlockSpec((BR, D_HID), lambda i: (i, 0)),
            pl.BlockSpec((BR, D_HID), lambda i: (i, 0)),
            pl.BlockSpec((BR, 1), lambda i: (i, 0)),
            pl.BlockSpec((1, D_HID), lambda i: (0, 0)),
            pl.BlockSpec((D_HID, N_CLASS), lambda i: (0, 0)),
        ],
        out_specs=pl.BlockSpec((BR, 2 * N_CLASS), lambda i: (i, 0)),
        out_shape=jax.ShapeDtypeStruct((N_PAD, 2 * N_CLASS), jnp.float32),
    )(p0, p1, xw1s, dis2d, b1, W2)


def _final_body(p0_ref, p1_ref, xws_ref, dis_ref, b2_ref, lsm_ref, sm_ref):
    s = (p0_ref[...] + p1_ref[...] + xws_ref[...])[:, :N_CLASS]
    logits = dis_ref[...] * s + b2_ref[...]
    m = jnp.max(logits, axis=1, keepdims=True)
    sh = logits - m
    ex = jnp.exp(sh)
    z = jnp.sum(ex, axis=1, keepdims=True)
    lsm_ref[...] = sh - jnp.log(z)
    sm_ref[...] = ex / z


def _final(p0, p1, xw2s, dis2d, b2):
    return pl.pallas_call(
        _final_body,
        grid=(N_PAD // BR,),
        in_specs=[
            pl.BlockSpec((BR, 2 * N_CLASS), lambda i: (i, 0)),
            pl.BlockSpec((BR, 2 * N_CLASS), lambda i: (i, 0)),
            pl.BlockSpec((BR, 2 * N_CLASS), lambda i: (i, 0)),
            pl.BlockSpec((BR, 1), lambda i: (i, 0)),
            pl.BlockSpec((1, N_CLASS), lambda i: (0, 0)),
        ],
        out_specs=[
            pl.BlockSpec((BR, N_CLASS), lambda i: (i, 0)),
            pl.BlockSpec((BR, N_CLASS), lambda i: (i, 0)),
        ],
        out_shape=[
            jax.ShapeDtypeStruct((N_PAD, N_CLASS), jnp.float32),
            jax.ShapeDtypeStruct((N_PAD, N_CLASS), jnp.float32),
        ],
    )(p0, p1, xw2s, dis2d, b2)


def _pack_edges(flat):
    pieces = []
    off = 0
    for w in range(NW):
        c = C0 if w % NC == 0 else C1
        seg = flat[off * CHUNK:(off + c) * CHUNK]
        if c < NCW:
            seg = jnp.concatenate(
                [seg, jnp.full(((NCW - c) * CHUNK,), PAD_NODE, jnp.int32)])
        pieces.append(seg)
        off += c
    return jnp.concatenate(pieces).reshape(NW, NCW, CHUNK)


def kernel(features, edge_index, W1, b1, W2, b2):
    src = edge_index[0].astype(jnp.int32)
    dst = edge_index[1].astype(jnp.int32)
    pad = jnp.full((E_PAD - E,), PAD_NODE, jnp.int32)
    src_p = _pack_edges(jnp.concatenate([src, pad]))
    dst_p = _pack_edges(jnp.concatenate([dst, pad]))
    x_pad = jnp.pad(features, ((0, N_PAD - N), (0, 0)))

    zo = jnp.stack([jnp.zeros((CHUNK,), jnp.float32),
                    jnp.ones((CHUNK,), jnp.float32)])
    deg = _deg_kernel(dst_p, zo).reshape(NC, N_PAD)
    d0 = deg[0].reshape(N_PAD, 1)
    d1 = deg[1].reshape(N_PAD, 1)

    xw1s, dis2d = _layer1(x_pad, W1, d0, d1)

    z_hid = jnp.zeros((CHUNK, D_HID), jnp.float32)
    s1 = _agg_hid(xw1s, src_p, dst_p, z_hid)

    xw2s = _layer2(s1[0], s1[1], xw1s, dis2d, b1.reshape(1, D_HID), W2)

    s2 = _agg_hid(xw2s, src_p, dst_p, z_hid)

    lsm, sm = _final(s2[0], s2[1], xw2s, dis2d, b2.reshape(1, N_CLASS))
    return lsm[:N], sm[:N]


# speedup vs baseline: 1.3499x; 1.0621x over previous
import functools

import jax
import jax.numpy as jnp
from jax import lax
from jax.experimental import pallas as pl
from jax.experimental.pallas import tpu as pltpu
from jax.experimental.pallas import tpu_sc as plsc

N = 10000
E = 320000
D_IN = 128
D_HID = 128
N_CLASS = 64

NC = 2
NS = 16
NW = NC * NS
CHUNK = 128
C0 = 112
C1 = 48
NCW = max(C0, C1)
N_USED_CHUNKS = NS * (C0 + C1)
E_PAD = N_USED_CHUNKS * CHUNK
PAD_NODE = N

N_PAD = 10112
ROWS_PER_TILE = N_PAD // NS
NZCH = ROWS_PER_TILE // CHUNK
TAIL = ROWS_PER_TILE - NZCH * CHUNK
BR = 128

_MESH = plsc.VectorSubcoreMesh(core_axis_name="c", subcore_axis_name="s")


def _make_agg(d):
    @functools.partial(
        pl.kernel,
        out_type=jax.ShapeDtypeStruct((NC, N_PAD, d), jnp.float32),
        mesh=_MESH,
        scratch_types=[
            pltpu.VMEM((8, CHUNK), jnp.int32),
            pltpu.VMEM((2, CHUNK, d), jnp.float32),
            pltpu.VMEM_SHARED((N_PAD, d), jnp.float32),
            pltpu.SemaphoreType.DMA,
            pltpu.SemaphoreType.DMA,
            pltpu.SemaphoreType.DMA,
            pltpu.SemaphoreType.DMA,
            pltpu.SemaphoreType.DMA,
        ],
    )
    def agg(xw_hbm, src_hbm, dst_hbm, zeros_hbm, out_hbm,
            ix_v, rows_v, acc_sh, gsem_a, gsem_b, isem_a, isem_b, csem):
        cid = lax.axis_index("c")
        sid = lax.axis_index("s")
        wid = sid * NC + cid
        base = sid * ROWS_PER_TILE

        pltpu.async_copy(zeros_hbm, rows_v.at[0], csem).wait()
        for r in range(NZCH):
            pltpu.sync_copy(rows_v.at[0], acc_sh.at[pl.ds(base + r * CHUNK, CHUNK)])
        pltpu.sync_copy(rows_v.at[0, pl.ds(0, TAIL)],
                        acc_sh.at[pl.ds(base + NZCH * CHUNK, TAIL)])

        plsc.subcore_barrier()

        cnt = jnp.where(cid == 0, C0, C1)

        @pl.loop(0, cnt, step=4)
        def _(j):
            hia = [pltpu.async_copy(src_hbm.at[wid, pl.ds(j, 2)],
                                    ix_v.at[pl.ds(0, 2)], isem_a),
                   pltpu.async_copy(dst_hbm.at[wid, pl.ds(j, 2)],
                                    ix_v.at[pl.ds(2, 2)], isem_a)]
            hib = [pltpu.async_copy(src_hbm.at[wid, pl.ds(j + 2, 2)],
                                    ix_v.at[pl.ds(4, 2)], isem_b),
                   pltpu.async_copy(dst_hbm.at[wid, pl.ds(j + 2, 2)],
                                    ix_v.at[pl.ds(6, 2)], isem_b)]
            for h in hia:
                h.wait()
            ga = pltpu.async_copy(xw_hbm.at[ix_v.at[0]], rows_v.at[0], gsem_a)
            ga.wait()
            gb = pltpu.async_copy(xw_hbm.at[ix_v.at[1]], rows_v.at[1], gsem_b)
            pltpu.sync_copy(rows_v.at[0], acc_sh.at[ix_v.at[2]], add=True)
            gb.wait()
            for h in hib:
                h.wait()
            ga = pltpu.async_copy(xw_hbm.at[ix_v.at[4]], rows_v.at[0], gsem_a)
            pltpu.sync_copy(rows_v.at[1], acc_sh.at[ix_v.at[3]], add=True)
            ga.wait()
            gb = pltpu.async_copy(xw_hbm.at[ix_v.at[5]], rows_v.at[1], gsem_b)
            pltpu.sync_copy(rows_v.at[0], acc_sh.at[ix_v.at[6]], add=True)
            gb.wait()
            pltpu.sync_copy(rows_v.at[1], acc_sh.at[ix_v.at[7]], add=True)

        plsc.subcore_barrier()

        ssems = (gsem_a, gsem_b)
        store_h = [None, None]
        for i, ln in enumerate([CHUNK] * NZCH + [TAIL]):
            b = i % 2
            off = base + i * CHUNK
            if store_h[b] is not None:
                store_h[b].wait()
            pltpu.async_copy(acc_sh.at[pl.ds(off, ln)],
                             rows_v.at[b, pl.ds(0, ln)], csem).wait()
            store_h[b] = pltpu.async_copy(rows_v.at[b, pl.ds(0, ln)],
                                          out_hbm.at[cid, pl.ds(off, ln)],
                                          ssems[b])
        for h in store_h:
            h.wait()

    return agg


_agg_hid = _make_agg(D_HID)


@functools.partial(
    pl.kernel,
    out_type=jax.ShapeDtypeStruct((NC * N_PAD,), jnp.float32),
    mesh=_MESH,
    scratch_types=[
        pltpu.VMEM((NCW, CHUNK), jnp.int32),
        pltpu.VMEM((2, CHUNK), jnp.float32),
        pltpu.VMEM((NZCH + 1, CHUNK), jnp.float32),
        pltpu.VMEM_SHARED((N_PAD,), jnp.float32),
        pltpu.SemaphoreType.DMA,
        pltpu.SemaphoreType.DMA,
    ],
)
def _deg_kernel(dst_hbm, zo_hbm, out_hbm, idx_v, zo_v, stage_v, acc_sh,
                ssem, csem):
    cid = lax.axis_index("c")
    sid = lax.axis_index("s")
    wid = sid * NC + cid

    h1 = pltpu.async_copy(dst_hbm.at[wid], idx_v, csem)
    h2 = pltpu.async_copy(zo_hbm, zo_v, csem)
    h1.wait()
    h2.wait()

    for r in range(NZCH):
        pltpu.sync_copy(zo_v.at[0],
                        acc_sh.at[pl.ds(sid * ROWS_PER_TILE + r * CHUNK, CHUNK)])
    pltpu.sync_copy(zo_v.at[0, pl.ds(0, TAIL)],
                    acc_sh.at[pl.ds(sid * ROWS_PER_TILE + NZCH * CHUNK, TAIL)])

    plsc.subcore_barrier()

    cnt = jnp.where(cid == 0, C0, C1)

    @pl.loop(0, cnt)
    def _(j):
        pltpu.async_copy(zo_v.at[1], acc_sh.at[idx_v.at[j]], ssem, add=True)

    @pl.loop(0, cnt)
    def _(j):
        pltpu.make_async_copy(zo_v.at[1], acc_sh.at[idx_v.at[0]], ssem).wait()

    plsc.subcore_barrier()

    tbase = sid * ROWS_PER_TILE + NZCH * CHUNK
    loads = [pltpu.async_copy(
                 acc_sh.at[pl.ds(sid * ROWS_PER_TILE + r * CHUNK, CHUNK)],
                 stage_v.at[r], csem) for r in range(NZCH)]
    loads.append(pltpu.async_copy(acc_sh.at[pl.ds(tbase, TAIL)],
                                  stage_v.at[NZCH, pl.ds(0, TAIL)], csem))
    for h in loads:
        h.wait()
    stores = [pltpu.async_copy(
                  stage_v.at[r],
                  out_hbm.at[pl.ds(cid * N_PAD + sid * ROWS_PER_TILE + r * CHUNK,
                                   CHUNK)],
                  csem) for r in range(NZCH)]
    stores.append(pltpu.async_copy(stage_v.at[NZCH, pl.ds(0, TAIL)],
                                   out_hbm.at[pl.ds(cid * N_PAD + tbase, TAIL)],
                                   csem))
    for h in stores:
        h.wait()


def _scale1_body(x_ref, w_ref, d0_ref, d1_ref, xws_ref, dis_ref):
    deg = d0_ref[...] + d1_ref[...] + 1.0
    dis = lax.rsqrt(deg)
    xw = jnp.dot(x_ref[...], w_ref[...], precision=lax.Precision.HIGHEST,
                 preferred_element_type=jnp.float32)
    xws_ref[...] = xw * dis
    dis_ref[...] = dis


def _layer1(x_pad, W1, d0, d1):
    return pl.pallas_call(
        _scale1_body,
        grid=(N_PAD // BR,),
        in_specs=[
            pl.BlockSpec((BR, D_IN), lambda i: (i, 0)),
            pl.BlockSpec((D_IN, D_HID), lambda i: (0, 0)),
            pl.BlockSpec((BR, 1), lambda i: (i, 0)),
            pl.BlockSpec((BR, 1), lambda i: (i, 0)),
        ],
        out_specs=[
            pl.BlockSpec((BR, D_HID), lambda i: (i, 0)),
            pl.BlockSpec((BR, 1), lambda i: (i, 0)),
        ],
        out_shape=[
            jax.ShapeDtypeStruct((N_PAD, D_HID), jnp.float32),
            jax.ShapeDtypeStruct((N_PAD, 1), jnp.float32),
        ],
    )(x_pad, W1, d0, d1)


def _layer2_body(p0_ref, p1_ref, xws_ref, dis_ref, b1_ref, w2_ref, out_ref):
    s = p0_ref[...] + p1_ref[...] + xws_ref[...]
    pre = dis_ref[...] * s + b1_ref[...]
    h = jnp.maximum(pre, 0.0)
    xw2 = jnp.dot(h, w2_ref[...], precision=lax.Precision.HIGHEST,
                  preferred_element_type=jnp.float32)
    xw2s = xw2 * dis_ref[...]
    out_ref[...] = jnp.concatenate([xw2s, jnp.zeros_like(xw2s)], axis=1)


def _layer2(p0, p1, xw1s, dis2d, b1, W2):
    return pl.pallas_call(
        _layer2_body,
        grid=(N_PAD // BR,),
        in_specs=[
            pl.BlockSpec((BR, D_HID), lambda i: (i, 0)),
            pl.BlockSpec((BR, D_HID), lambda i: (i, 0)),
            pl.BlockSpec((BR, D_HID), lambda i: (i, 0)),
            pl.BlockSpec((BR, 1), lambda i: (i, 0)),
            pl.BlockSpec((1, D_HID), lambda i: (0, 0)),
            pl.BlockSpec((D_HID, N_CLASS), lambda i: (0, 0)),
        ],
        out_specs=pl.BlockSpec((BR, 2 * N_CLASS), lambda i: (i, 0)),
        out_shape=jax.ShapeDtypeStruct((N_PAD, 2 * N_CLASS), jnp.float32),
    )(p0, p1, xw1s, dis2d, b1, W2)


def _final_body(p0_ref, p1_ref, xws_ref, dis_ref, b2_ref, lsm_ref, sm_ref):
    s = (p0_ref[...] + p1_ref[...] + xws_ref[...])[:, :N_CLASS]
    logits = dis_ref[...] * s + b2_ref[...]
    m = jnp.max(logits, axis=1, keepdims=True)
    sh = logits - m
    ex = jnp.exp(sh)
    z = jnp.sum(ex, axis=1, keepdims=True)
    lsm_ref[...] = sh - jnp.log(z)
    sm_ref[...] = ex / z


def _final(p0, p1, xw2s, dis2d, b2):
    return pl.pallas_call(
        _final_body,
        grid=(N_PAD // BR,),
        in_specs=[
            pl.BlockSpec((BR, 2 * N_CLASS), lambda i: (i, 0)),
            pl.BlockSpec((BR, 2 * N_CLASS), lambda i: (i, 0)),
            pl.BlockSpec((BR, 2 * N_CLASS), lambda i: (i, 0)),
            pl.BlockSpec((BR, 1), lambda i: (i, 0)),
            pl.BlockSpec((1, N_CLASS), lambda i: (0, 0)),
        ],
        out_specs=[
            pl.BlockSpec((BR, N_CLASS), lambda i: (i, 0)),
            pl.BlockSpec((BR, N_CLASS), lambda i: (i, 0)),
        ],
        out_shape=[
            jax.ShapeDtypeStruct((N_PAD, N_CLASS), jnp.float32),
            jax.ShapeDtypeStruct((N_PAD, N_CLASS), jnp.float32),
        ],
    )(p0, p1, xw2s, dis2d, b2)


def _pack_edges(flat):
    pieces = []
    off = 0
    for w in range(NW):
        c = C0 if w % NC == 0 else C1
        seg = flat[off * CHUNK:(off + c) * CHUNK]
        if c < NCW:
            seg = jnp.concatenate(
                [seg, jnp.full(((NCW - c) * CHUNK,), PAD_NODE, jnp.int32)])
        pieces.append(seg)
        off += c
    return jnp.concatenate(pieces).reshape(NW, NCW, CHUNK)


def kernel(features, edge_index, W1, b1, W2, b2):
    src = edge_index[0].astype(jnp.int32)
    dst = edge_index[1].astype(jnp.int32)
    pad = jnp.full((E_PAD - E,), PAD_NODE, jnp.int32)
    src_p = _pack_edges(jnp.concatenate([src, pad]))
    dst_p = _pack_edges(jnp.concatenate([dst, pad]))
    x_pad = jnp.pad(features, ((0, N_PAD - N), (0, 0)))

    zo = jnp.stack([jnp.zeros((CHUNK,), jnp.float32),
                    jnp.ones((CHUNK,), jnp.float32)])
    deg = _deg_kernel(dst_p, zo).reshape(NC, N_PAD)
    d0 = deg[0].reshape(N_PAD, 1)
    d1 = deg[1].reshape(N_PAD, 1)

    xw1s, dis2d = _layer1(x_pad, W1, d0, d1)

    z_hid = jnp.zeros((CHUNK, D_HID), jnp.float32)
    s1 = _agg_hid(xw1s, src_p, dst_p, z_hid)

    xw2s = _layer2(s1[0], s1[1], xw1s, dis2d, b1.reshape(1, D_HID), W2)

    s2 = _agg_hid(xw2s, src_p, dst_p, z_hid)

    lsm, sm = _final(s2[0], s2[1], xw2s, dis2d, b2.reshape(1, N_CLASS))
    return lsm[:N], sm[:N]
